# Initial kernel scaffold; baseline (speedup 1.0000x reference)
import jax
import jax.numpy as jnp
from jax.experimental import pallas as pl


def kernel(xyz, features, fc1_w, fc1_b, fc2_w, fc2_b, delta_w1, delta_b1, delta_w2, delta_b2, gamma_w1, gamma_b1, gamma_w2, gamma_b2, wq, wk, wv, sim_w, sim_b):
    raise NotImplementedError("write your pallas kernel here")


# speedup vs baseline: 8.0790x; 8.0790x over previous
import functools
import math

import jax
import jax.numpy as jnp
from jax.experimental import pallas as pl
from jax.experimental.pallas import tpu as pltpu
from jax.experimental.pallas import tpu_sc as plsc

B, N = 2, 2048
DP, DM, K = 128, 512, 16
BN = B * N
NIDX = BN * K

BLKP = 512
BLKQ = 256
BLKH = 64
_HI = jax.lax.Precision.HIGHEST


def _preproc(features, fc1_wT, fc1_b, wqT, wkT, wvT):
    def body(f_ref, w1_ref, b1_ref, wq_ref, wk_ref, wv_ref,
             q_ref, k_ref, v_ref):
        x = jnp.dot(f_ref[...], w1_ref[...], precision=_HI) + b1_ref[...]
        q_ref[...] = jnp.dot(x, wq_ref[...], precision=_HI)
        k_ref[...] = jnp.dot(x, wk_ref[...], precision=_HI)
        v_ref[...] = jnp.dot(x, wv_ref[...], precision=_HI)

    grid = (BN // BLKP,)
    return pl.pallas_call(
        body,
        grid=grid,
        in_specs=[
            pl.BlockSpec((BLKP, DP), lambda i: (i, 0)),
            pl.BlockSpec((DP, DM), lambda i: (0, 0)),
            pl.BlockSpec((1, DM), lambda i: (0, 0)),
            pl.BlockSpec((DM, DM), lambda i: (0, 0)),
            pl.BlockSpec((DM, DM), lambda i: (0, 0)),
            pl.BlockSpec((DM, DM), lambda i: (0, 0)),
        ],
        out_specs=[
            pl.BlockSpec((BLKP, DM), lambda i: (i, 0)),
            pl.BlockSpec((BLKP, DM), lambda i: (i, 0)),
            pl.BlockSpec((BLKP, DM), lambda i: (i, 0)),
        ],
        out_shape=[
            jax.ShapeDtypeStruct((BN, DM), jnp.float32),
            jax.ShapeDtypeStruct((BN, DM), jnp.float32),
            jax.ShapeDtypeStruct((BN, DM), jnp.float32),
        ],
    )(features, fc1_wT, fc1_b, wqT, wkT, wvT)


def _knn(xyz8, xyz_all_t):
    nb = N // BLKQ

    def body(xq_ref, xa_ref, idx_ref):
        g = pl.program_id(0)
        b = g // nb
        xq = xq_ref[...]
        xa = xa_ref[0]
        qs2 = jnp.sum(xq * xq, axis=1, keepdims=True)
        ps2 = jnp.sum(xa * xa, axis=0, keepdims=True)
        qp = jnp.dot(xq, xa, precision=jax.lax.Precision.DEFAULT)
        dist = (qs2 + ps2) - 2.0 * qp
        iota = jax.lax.broadcasted_iota(jnp.int32, (BLKQ, N), 1)
        kiota = jax.lax.broadcasted_iota(jnp.int32, (BLKQ, K), 1)
        sel = jnp.zeros((BLKQ, K), jnp.int32)
        d = dist
        for j in range(K):
            mv = jnp.min(d, axis=1, keepdims=True)
            idx = jnp.min(jnp.where(d == mv, iota, N), axis=1, keepdims=True)
            sel = jnp.where(kiota == j, idx, sel)
            d = jnp.where(iota == idx, jnp.inf, d)
        idx_ref[...] = sel + b * N

    return pl.pallas_call(
        body,
        grid=(BN // BLKQ,),
        in_specs=[
            pl.BlockSpec((BLKQ, 8), lambda g: (g, 0)),
            pl.BlockSpec((1, 8, N), lambda g: (g // nb, 0, 0)),
        ],
        out_specs=pl.BlockSpec((BLKQ, K), lambda g: (g, 0)),
        out_shape=jax.ShapeDtypeStruct((BN, K), jnp.int32),
    )(xyz8, xyz_all_t)


def _gather_sc(kf, vf, xg, idx_flat):
    NC, NS = 2, 16
    NW = NC * NS
    BPW = NIDX // NW
    C = 64
    S = BPW // C
    mesh = plsc.VectorSubcoreMesh(core_axis_name="c", subcore_axis_name="s")

    @functools.partial(
        pl.kernel,
        out_type=(
            jax.ShapeDtypeStruct((NIDX, DM), jnp.float32),
            jax.ShapeDtypeStruct((NIDX, DM), jnp.float32),
            jax.ShapeDtypeStruct((NIDX, 128), jnp.float32),
        ),
        mesh=mesh,
        scratch_types=[
            pltpu.VMEM((C,), jnp.int32),
            pltpu.VMEM((C, DM), jnp.float32),
            pltpu.VMEM((C, DM), jnp.float32),
            pltpu.VMEM((C, 128), jnp.float32),
            pltpu.SemaphoreType.DMA,
            pltpu.SemaphoreType.DMA,
            pltpu.SemaphoreType.DMA,
        ],
    )
    def kern(kf_hbm, vf_hbm, xg_hbm, idx_hbm, ko_hbm, vo_hbm, xo_hbm,
             idxc, kbuf, vbuf, xbuf, sk, sv, sx):
        wid = jax.lax.axis_index("s") * NC + jax.lax.axis_index("c")
        base = wid * BPW

        @pl.loop(0, S)
        def _(s):
            off = base + s * C
            pltpu.sync_copy(idx_hbm.at[pl.ds(off, C)], idxc)
            ck = pltpu.async_copy(kf_hbm.at[idxc], kbuf, sk)
            cv = pltpu.async_copy(vf_hbm.at[idxc], vbuf, sv)
            cx = pltpu.async_copy(xg_hbm.at[idxc], xbuf, sx)
            ck.wait()
            cv.wait()
            cx.wait()
            pltpu.sync_copy(kbuf, ko_hbm.at[pl.ds(off, C)])
            pltpu.sync_copy(vbuf, vo_hbm.at[pl.ds(off, C)])
            pltpu.sync_copy(xbuf, xo_hbm.at[pl.ds(off, C)])

    return kern(kf, vf, xg, idx_flat)


def _heavy(kg, vg, xg, q, xq8, pre,
           dw1T, db1, dw2T, db2, sw0, swT, sim_b,
           gw1T, gb1, gw2T, gb2, fc2T, fc2_b):
    M = BLKH * K
    inv_scale = 1.0 / math.sqrt(float(DM))

    def body(kg_ref, vg_ref, xg_ref, q_ref, xq_ref, pre_ref,
             dw1_ref, db1_ref, dw2_ref, db2_ref, sw0_ref, swT_ref, sb_ref,
             gw1_ref, gb1_ref, gw2_ref, gb2_ref, fc2_ref, fb_ref,
             attn_ref, res_ref):
        kgv = kg_ref[...]
        q = q_ref[...]
        qn = jnp.maximum(jnp.sqrt(jnp.sum(q * q, axis=1, keepdims=True)), 1e-8)
        qe = jnp.broadcast_to(q[:, None, :], (BLKH, K, DM)).reshape(M, DM)
        qne = jnp.broadcast_to(qn[:, None, :], (BLKH, K, 1)).reshape(M, 1)
        xe = jnp.broadcast_to(xq_ref[...][:, None, :], (BLKH, K, 128)).reshape(M, 128)

        d = (xe - xg_ref[...]).astype(jnp.bfloat16)
        s1 = jnp.maximum(
            jnp.dot(d, dw1_ref[...], preferred_element_type=jnp.float32)
            + db1_ref[...], 0.0).astype(jnp.bfloat16)
        pos = (jnp.dot(s1, dw2_ref[...], preferred_element_type=jnp.float32)
               + db2_ref[...])

        kf32 = kgv
        num = jnp.sum(qe * kf32, axis=1, keepdims=True)
        kn = jnp.maximum(jnp.sqrt(jnp.sum(kf32 * kf32, axis=1, keepdims=True)), 1e-8)
        sim = num / (qne * kn)

        qmk = (qe - kf32).astype(jnp.bfloat16)
        rel = (sim * sw0_ref[...]
               + jnp.dot(qmk, swT_ref[...], preferred_element_type=jnp.float32)
               + sb_ref[...])
        h = (rel + pos).astype(jnp.bfloat16)
        a1 = jnp.maximum(
            jnp.dot(h, gw1_ref[...], preferred_element_type=jnp.float32)
            + gb1_ref[...], 0.0).astype(jnp.bfloat16)
        logits = (jnp.dot(a1, gw2_ref[...], preferred_element_type=jnp.float32)
                  + gb2_ref[...])
        l3 = (logits * inv_scale).reshape(BLKH, K, DM)
        m = jnp.max(l3, axis=1, keepdims=True)
        e = jnp.exp(l3 - m)
        s = jnp.sum(e, axis=1, keepdims=True)
        attn3 = e / s
        attn_ref[...] = attn3.reshape(M, DM)

        ve = (vg_ref[...] + pos).reshape(BLKH, K, DM)
        res = jnp.sum(attn3 * ve, axis=1)
        res_ref[...] = (jnp.dot(res, fc2_ref[...], precision=_HI)
                        + fb_ref[...] + pre_ref[...])

    grid = (BN // BLKH,)
    wspec = lambda shape: pl.BlockSpec(shape, lambda i: tuple(0 for _ in shape))
    return pl.pallas_call(
        body,
        grid=grid,
        in_specs=[
            pl.BlockSpec((M, DM), lambda i: (i, 0)),
            pl.BlockSpec((M, DM), lambda i: (i, 0)),
            pl.BlockSpec((M, 128), lambda i: (i, 0)),
            pl.BlockSpec((BLKH, DM), lambda i: (i, 0)),
            pl.BlockSpec((BLKH, 128), lambda i: (i, 0)),
            pl.BlockSpec((BLKH, DP), lambda i: (i, 0)),
            wspec((128, DM)), wspec((1, DM)), wspec((DM, DM)), wspec((1, DM)),
            wspec((1, DM)), wspec((DM, DM)), wspec((1, DM)),
            wspec((DM, DM)), wspec((1, DM)), wspec((DM, DM)), wspec((1, DM)),
            wspec((DM, DP)), wspec((1, DP)),
        ],
        out_specs=[
            pl.BlockSpec((M, DM), lambda i: (i, 0)),
            pl.BlockSpec((BLKH, DP), lambda i: (i, 0)),
        ],
        out_shape=[
            jax.ShapeDtypeStruct((NIDX, DM), jnp.float32),
            jax.ShapeDtypeStruct((BN, DP), jnp.float32),
        ],
    )(kg, vg, xg, q, xq8, pre,
      dw1T, db1, dw2T, db2, sw0, swT, sim_b,
      gw1T, gb1, gw2T, gb2, fc2T, fc2_b)


def kernel(xyz, features, fc1_w, fc1_b, fc2_w, fc2_b, delta_w1, delta_b1,
           delta_w2, delta_b2, gamma_w1, gamma_b1, gamma_w2, gamma_b2,
           wq, wk, wv, sim_w, sim_b):
    feats = features.reshape(BN, DP)
    q, kf, vf = _preproc(feats, fc1_w.T, fc1_b[None, :],
                         wq.T, wk.T, wv.T)

    xyz_flat = xyz.reshape(BN, 3)
    xyz8 = jnp.pad(xyz_flat, ((0, 0), (0, 5)))
    xyz_all_t = jnp.pad(xyz, ((0, 0), (0, 0), (0, 5))).transpose(0, 2, 1)
    knn_flat = _knn(xyz8, xyz_all_t)

    xg_tab = jnp.pad(xyz_flat, ((0, 0), (0, 125)))
    kg, vg, xg = _gather_sc(kf, vf, xg_tab, knn_flat.reshape(NIDX))

    dw1T = jnp.pad(delta_w1.T, ((0, 125), (0, 0))).astype(jnp.bfloat16)
    attn_flat, res_flat = _heavy(
        kg, vg, xg, q, xg_tab, feats,
        dw1T, delta_b1[None, :],
        delta_w2.T.astype(jnp.bfloat16), delta_b2[None, :],
        sim_w[:, 0][None, :], sim_w[:, 1:].T.astype(jnp.bfloat16),
        sim_b[None, :],
        gamma_w1.T.astype(jnp.bfloat16), gamma_b1[None, :],
        gamma_w2.T.astype(jnp.bfloat16), gamma_b2[None, :],
        fc2_w.T, fc2_b[None, :])

    res = res_flat.reshape(B, N, DP)
    attn = attn_flat.reshape(B, N, K, DM)
    return (res, attn)


# speedup vs baseline: 9.3358x; 1.1556x over previous
import functools
import math

import jax
import jax.numpy as jnp
from jax.experimental import pallas as pl
from jax.experimental.pallas import tpu as pltpu
from jax.experimental.pallas import tpu_sc as plsc

B, N = 2, 2048
DP, DM, K = 128, 512, 16
BN = B * N
NIDX = BN * K

BLKP = 512
BLKQ = 256
BLKH = 64
_HI = jax.lax.Precision.HIGHEST


def _preproc(features, fc1_wT, fc1_b, wqT, wkT, wvT):
    def body(f_ref, w1_ref, b1_ref, wq_ref, wk_ref, wv_ref,
             q_ref, kv_ref):
        x = jnp.dot(f_ref[...], w1_ref[...], precision=_HI) + b1_ref[...]
        q_ref[...] = jnp.dot(x, wq_ref[...], precision=_HI)
        k = jnp.dot(x, wk_ref[...], precision=_HI)
        v = jnp.dot(x, wv_ref[...], precision=_HI)

        def _rne_hi(t):
            u = jax.lax.bitcast_convert_type(t, jnp.uint32)
            r = u + jnp.uint32(0x7FFF) + ((u >> 16) & jnp.uint32(1))
            return r & jnp.uint32(0xFFFF0000)

        packed = _rne_hi(k) | (_rne_hi(v) >> 16)
        kv_ref[...] = jax.lax.bitcast_convert_type(packed, jnp.int32)

    grid = (BN // BLKP,)
    return pl.pallas_call(
        body,
        grid=grid,
        in_specs=[
            pl.BlockSpec((BLKP, DP), lambda i: (i, 0)),
            pl.BlockSpec((DP, DM), lambda i: (0, 0)),
            pl.BlockSpec((1, DM), lambda i: (0, 0)),
            pl.BlockSpec((DM, DM), lambda i: (0, 0)),
            pl.BlockSpec((DM, DM), lambda i: (0, 0)),
            pl.BlockSpec((DM, DM), lambda i: (0, 0)),
        ],
        out_specs=[
            pl.BlockSpec((BLKP, DM), lambda i: (i, 0)),
            pl.BlockSpec((BLKP, DM), lambda i: (i, 0)),
        ],
        out_shape=[
            jax.ShapeDtypeStruct((BN, DM), jnp.float32),
            jax.ShapeDtypeStruct((BN, DM), jnp.int32),
        ],
    )(features, fc1_wT, fc1_b, wqT, wkT, wvT)


def _knn(xyz8, xyz_all_t):
    nb = N // BLKQ

    def body(xq_ref, xa_ref, idx_ref):
        g = pl.program_id(0)
        b = g // nb
        xq = xq_ref[...]
        xa = xa_ref[0]
        qs2 = jnp.sum(xq * xq, axis=1, keepdims=True)
        ps2 = jnp.sum(xa * xa, axis=0, keepdims=True)
        qp = jnp.dot(xq, xa, precision=jax.lax.Precision.DEFAULT)
        dist = (qs2 + ps2) - 2.0 * qp
        iota = jax.lax.broadcasted_iota(jnp.int32, (BLKQ, N), 1)
        kiota = jax.lax.broadcasted_iota(jnp.int32, (BLKQ, K), 1)
        sel = jnp.zeros((BLKQ, K), jnp.int32)
        d = dist
        for j in range(K):
            mv = jnp.min(d, axis=1, keepdims=True)
            idx = jnp.min(jnp.where(d == mv, iota, N), axis=1, keepdims=True)
            sel = jnp.where(kiota == j, idx, sel)
            d = jnp.where(iota == idx, jnp.inf, d)
        idx_ref[...] = sel + b * N

    return pl.pallas_call(
        body,
        grid=(BN // BLKQ,),
        in_specs=[
            pl.BlockSpec((BLKQ, 8), lambda g: (g, 0)),
            pl.BlockSpec((1, 8, N), lambda g: (g // nb, 0, 0)),
        ],
        out_specs=pl.BlockSpec((BLKQ, K), lambda g: (g, 0)),
        out_shape=jax.ShapeDtypeStruct((BN, K), jnp.int32),
    )(xyz8, xyz_all_t)


def _gather_sc(kv, xg, idx_flat):
    NC, NS = 2, 16
    NW = NC * NS
    BPW = NIDX // NW
    C = 128
    S = BPW // C
    mesh = plsc.VectorSubcoreMesh(core_axis_name="c", subcore_axis_name="s")

    @functools.partial(
        pl.kernel,
        out_type=(
            jax.ShapeDtypeStruct((NIDX, DM), jnp.int32),
            jax.ShapeDtypeStruct((NIDX, 128), jnp.float32),
        ),
        mesh=mesh,
        scratch_types=[
            pltpu.VMEM((C,), jnp.int32),
            pltpu.VMEM((C, DM), jnp.int32),
            pltpu.VMEM((C, 128), jnp.float32),
            pltpu.SemaphoreType.DMA,
            pltpu.SemaphoreType.DMA,
        ],
    )
    def kern(kv_hbm, xg_hbm, idx_hbm, kvo_hbm, xo_hbm,
             idxc, kvbuf, xbuf, sk, sx):
        wid = jax.lax.axis_index("s") * NC + jax.lax.axis_index("c")
        base = wid * BPW

        @pl.loop(0, S)
        def _(s):
            off = base + s * C
            pltpu.sync_copy(idx_hbm.at[pl.ds(off, C)], idxc)
            ck = pltpu.async_copy(kv_hbm.at[idxc], kvbuf, sk)
            cx = pltpu.async_copy(xg_hbm.at[idxc], xbuf, sx)
            ck.wait()
            cx.wait()
            pltpu.sync_copy(kvbuf, kvo_hbm.at[pl.ds(off, C)])
            pltpu.sync_copy(xbuf, xo_hbm.at[pl.ds(off, C)])

    return kern(kv, xg, idx_flat)


def _heavy(kvg, xg, q, xq8, pre,
           dw1T, db1, dw2T, db2, sw0, swT, sim_b,
           gw1T, gb1, gw2T, gb2, fc2T, fc2_b):
    M = BLKH * K
    inv_scale = 1.0 / math.sqrt(float(DM))

    def body(kv_ref, xg_ref, q_ref, xq_ref, pre_ref,
             dw1_ref, db1_ref, dw2_ref, db2_ref, sw0_ref, swT_ref, sb_ref,
             gw1_ref, gb1_ref, gw2_ref, gb2_ref, fc2_ref, fb_ref,
             attn_ref, res_ref):
        ku = jax.lax.bitcast_convert_type(kv_ref[...], jnp.uint32)
        kf32 = jax.lax.bitcast_convert_type(ku & jnp.uint32(0xFFFF0000),
                                            jnp.float32)
        vf32 = jax.lax.bitcast_convert_type(ku << 16, jnp.float32)
        q = q_ref[...]
        qn = jnp.maximum(jnp.sqrt(jnp.sum(q * q, axis=1, keepdims=True)), 1e-8)
        qe = jnp.broadcast_to(q[:, None, :], (BLKH, K, DM)).reshape(M, DM)
        qne = jnp.broadcast_to(qn[:, None, :], (BLKH, K, 1)).reshape(M, 1)
        xe = jnp.broadcast_to(xq_ref[...][:, None, :], (BLKH, K, 128)).reshape(M, 128)

        d = (xe - xg_ref[...]).astype(jnp.bfloat16)
        s1 = jnp.maximum(
            jnp.dot(d, dw1_ref[...], preferred_element_type=jnp.float32)
            + db1_ref[...], 0.0).astype(jnp.bfloat16)
        pos = (jnp.dot(s1, dw2_ref[...], preferred_element_type=jnp.float32)
               + db2_ref[...])

        num = jnp.sum(qe * kf32, axis=1, keepdims=True)
        kn = jnp.maximum(jnp.sqrt(jnp.sum(kf32 * kf32, axis=1, keepdims=True)), 1e-8)
        sim = num / (qne * kn)

        qmk = (qe - kf32).astype(jnp.bfloat16)
        rel = (sim * sw0_ref[...]
               + jnp.dot(qmk, swT_ref[...], preferred_element_type=jnp.float32)
               + sb_ref[...])
        h = (rel + pos).astype(jnp.bfloat16)
        a1 = jnp.maximum(
            jnp.dot(h, gw1_ref[...], preferred_element_type=jnp.float32)
            + gb1_ref[...], 0.0).astype(jnp.bfloat16)
        logits = (jnp.dot(a1, gw2_ref[...], preferred_element_type=jnp.float32)
                  + gb2_ref[...])
        l3 = (logits * inv_scale).reshape(BLKH, K, DM)
        m = jnp.max(l3, axis=1, keepdims=True)
        e = jnp.exp(l3 - m)
        s = jnp.sum(e, axis=1, keepdims=True)
        attn3 = e / s
        attn_ref[...] = attn3.reshape(M, DM)

        ve = (vf32 + pos).reshape(BLKH, K, DM)
        res = jnp.sum(attn3 * ve, axis=1)
        res_ref[...] = (jnp.dot(res, fc2_ref[...], precision=_HI)
                        + fb_ref[...] + pre_ref[...])

    grid = (BN // BLKH,)
    wspec = lambda shape: pl.BlockSpec(shape, lambda i: tuple(0 for _ in shape))
    return pl.pallas_call(
        body,
        grid=grid,
        in_specs=[
            pl.BlockSpec((M, DM), lambda i: (i, 0)),
            pl.BlockSpec((M, 128), lambda i: (i, 0)),
            pl.BlockSpec((BLKH, DM), lambda i: (i, 0)),
            pl.BlockSpec((BLKH, 128), lambda i: (i, 0)),
            pl.BlockSpec((BLKH, DP), lambda i: (i, 0)),
            wspec((128, DM)), wspec((1, DM)), wspec((DM, DM)), wspec((1, DM)),
            wspec((1, DM)), wspec((DM, DM)), wspec((1, DM)),
            wspec((DM, DM)), wspec((1, DM)), wspec((DM, DM)), wspec((1, DM)),
            wspec((DM, DP)), wspec((1, DP)),
        ],
        out_specs=[
            pl.BlockSpec((M, DM), lambda i: (i, 0)),
            pl.BlockSpec((BLKH, DP), lambda i: (i, 0)),
        ],
        out_shape=[
            jax.ShapeDtypeStruct((NIDX, DM), jnp.float32),
            jax.ShapeDtypeStruct((BN, DP), jnp.float32),
        ],
    )(kvg, xg, q, xq8, pre,
      dw1T, db1, dw2T, db2, sw0, swT, sim_b,
      gw1T, gb1, gw2T, gb2, fc2T, fc2_b)


def kernel(xyz, features, fc1_w, fc1_b, fc2_w, fc2_b, delta_w1, delta_b1,
           delta_w2, delta_b2, gamma_w1, gamma_b1, gamma_w2, gamma_b2,
           wq, wk, wv, sim_w, sim_b):
    feats = features.reshape(BN, DP)
    q, kvtab = _preproc(feats, fc1_w.T, fc1_b[None, :],
                        wq.T, wk.T, wv.T)

    xyz_flat = xyz.reshape(BN, 3)
    xyz8 = jnp.pad(xyz_flat, ((0, 0), (0, 5)))
    xyz_all_t = jnp.pad(xyz, ((0, 0), (0, 0), (0, 5))).transpose(0, 2, 1)
    knn_flat = _knn(xyz8, xyz_all_t)

    xg_tab = jnp.pad(xyz_flat, ((0, 0), (0, 125)))
    kvg, xg = _gather_sc(kvtab, xg_tab, knn_flat.reshape(NIDX))

    dw1T = jnp.pad(delta_w1.T, ((0, 125), (0, 0))).astype(jnp.bfloat16)
    attn_flat, res_flat = _heavy(
        kvg, xg, q, xg_tab, feats,
        dw1T, delta_b1[None, :],
        delta_w2.T.astype(jnp.bfloat16), delta_b2[None, :],
        sim_w[:, 0][None, :], sim_w[:, 1:].T.astype(jnp.bfloat16),
        sim_b[None, :],
        gamma_w1.T.astype(jnp.bfloat16), gamma_b1[None, :],
        gamma_w2.T.astype(jnp.bfloat16), gamma_b2[None, :],
        fc2_w.T, fc2_b[None, :])

    res = res_flat.reshape(B, N, DP)
    attn = attn_flat.reshape(B, N, K, DM)
    return (res, attn)


# speedup vs baseline: 9.7805x; 1.0476x over previous
import functools
import math

import jax
import jax.numpy as jnp
from jax.experimental import pallas as pl
from jax.experimental.pallas import tpu as pltpu
from jax.experimental.pallas import tpu_sc as plsc

B, N = 2, 2048
DP, DM, K = 128, 512, 16
BN = B * N
NIDX = BN * K

BLKP = 512
BLKQ = 256
BLKH = 64
_HI = jax.lax.Precision.HIGHEST


def _preproc(features, fc1_wT, fc1_b, wqT, wkT, wvT):
    def body(f_ref, w1_ref, b1_ref, wq_ref, wk_ref, wv_ref,
             q_ref, kv_ref):
        x = jnp.dot(f_ref[...], w1_ref[...]) + b1_ref[...]
        q_ref[...] = jnp.dot(x, wq_ref[...])
        k = jnp.dot(x, wk_ref[...])
        v = jnp.dot(x, wv_ref[...])

        def _rne_hi(t):
            u = jax.lax.bitcast_convert_type(t, jnp.uint32)
            r = u + jnp.uint32(0x7FFF) + ((u >> 16) & jnp.uint32(1))
            return r & jnp.uint32(0xFFFF0000)

        packed = _rne_hi(k) | (_rne_hi(v) >> 16)
        kv_ref[...] = jax.lax.bitcast_convert_type(packed, jnp.int32)

    grid = (BN // BLKP,)
    return pl.pallas_call(
        body,
        grid=grid,
        in_specs=[
            pl.BlockSpec((BLKP, DP), lambda i: (i, 0)),
            pl.BlockSpec((DP, DM), lambda i: (0, 0)),
            pl.BlockSpec((1, DM), lambda i: (0, 0)),
            pl.BlockSpec((DM, DM), lambda i: (0, 0)),
            pl.BlockSpec((DM, DM), lambda i: (0, 0)),
            pl.BlockSpec((DM, DM), lambda i: (0, 0)),
        ],
        out_specs=[
            pl.BlockSpec((BLKP, DM), lambda i: (i, 0)),
            pl.BlockSpec((BLKP, DM), lambda i: (i, 0)),
        ],
        out_shape=[
            jax.ShapeDtypeStruct((BN, DM), jnp.float32),
            jax.ShapeDtypeStruct((BN, DM), jnp.int32),
        ],
    )(features, fc1_wT, fc1_b, wqT, wkT, wvT)


def _knn(xyz8, xyz_all_t, base_row):

    def body(xq_ref, xa_ref, idx_ref):
        xq = xq_ref[...]
        xa = xa_ref[0]
        qs2 = jnp.sum(xq * xq, axis=1, keepdims=True)
        ps2 = jnp.sum(xa * xa, axis=0, keepdims=True)
        qp = jnp.dot(xq, xa, precision=jax.lax.Precision.DEFAULT)
        dist = (qs2 + ps2) - 2.0 * qp
        iota = jax.lax.broadcasted_iota(jnp.int32, (BLKQ, N), 1)
        kiota = jax.lax.broadcasted_iota(jnp.int32, (BLKQ, K), 1)
        sel = jnp.zeros((BLKQ, K), jnp.int32)
        d = dist
        for j in range(K):
            mv = jnp.min(d, axis=1, keepdims=True)
            idx = jnp.min(jnp.where(d == mv, iota, N), axis=1, keepdims=True)
            sel = jnp.where(kiota == j, idx, sel)
            d = jnp.where(iota == idx, jnp.inf, d)
        idx_ref[...] = sel + base_row

    return pl.pallas_call(
        body,
        grid=(N // BLKQ,),
        in_specs=[
            pl.BlockSpec((BLKQ, 8), lambda g: (g, 0)),
            pl.BlockSpec((1, 8, N), lambda g: (0, 0, 0)),
        ],
        out_specs=pl.BlockSpec((BLKQ, K), lambda g: (g, 0)),
        out_shape=jax.ShapeDtypeStruct((N, K), jnp.int32),
    )(xyz8, xyz_all_t)


def _gather_sc(kv, xg, idx_flat):
    NI = idx_flat.shape[0]
    NC, NS = 2, 16
    NW = NC * NS
    BPW = NI // NW
    C = 128
    S = BPW // C
    mesh = plsc.VectorSubcoreMesh(core_axis_name="c", subcore_axis_name="s")

    @functools.partial(
        pl.kernel,
        out_type=(
            jax.ShapeDtypeStruct((NI, DM), jnp.int32),
            jax.ShapeDtypeStruct((NI, 128), jnp.float32),
        ),
        mesh=mesh,
        scratch_types=[
            pltpu.VMEM((C,), jnp.int32),
            pltpu.VMEM((C, DM), jnp.int32),
            pltpu.VMEM((C, 128), jnp.float32),
            pltpu.SemaphoreType.DMA,
            pltpu.SemaphoreType.DMA,
        ],
    )
    def kern(kv_hbm, xg_hbm, idx_hbm, kvo_hbm, xo_hbm,
             idxc, kvbuf, xbuf, sk, sx):
        wid = jax.lax.axis_index("s") * NC + jax.lax.axis_index("c")
        base = wid * BPW

        @pl.loop(0, S)
        def _(s):
            off = base + s * C
            pltpu.sync_copy(idx_hbm.at[pl.ds(off, C)], idxc)
            ck = pltpu.async_copy(kv_hbm.at[idxc], kvbuf, sk)
            cx = pltpu.async_copy(xg_hbm.at[idxc], xbuf, sx)
            ck.wait()
            cx.wait()
            pltpu.sync_copy(kvbuf, kvo_hbm.at[pl.ds(off, C)])
            pltpu.sync_copy(xbuf, xo_hbm.at[pl.ds(off, C)])

    return kern(kv, xg, idx_flat)


def _heavy(kvg, xg, q, xq8, pre,
           dw1T, db1, dw2T, db2, sw0, swT, sim_b,
           gw1T, gb1, gw2T, gb2, fc2T, fc2_b):
    NR = q.shape[0]
    M = BLKH * K
    inv_scale = 1.0 / math.sqrt(float(DM))

    def body(kv_ref, xg_ref, q_ref, xq_ref, pre_ref,
             dw1_ref, db1_ref, dw2_ref, db2_ref, sw0_ref, swT_ref, sb_ref,
             gw1_ref, gb1_ref, gw2_ref, gb2_ref, fc2_ref, fb_ref,
             attn_ref, res_ref):
        ku = jax.lax.bitcast_convert_type(kv_ref[...], jnp.uint32)
        kf32 = jax.lax.bitcast_convert_type(ku & jnp.uint32(0xFFFF0000),
                                            jnp.float32)
        vf32 = jax.lax.bitcast_convert_type(ku << 16, jnp.float32)
        q = q_ref[...]
        qn = jnp.maximum(jnp.sqrt(jnp.sum(q * q, axis=1, keepdims=True)), 1e-8)
        qe = jnp.broadcast_to(q[:, None, :], (BLKH, K, DM)).reshape(M, DM)
        qne = jnp.broadcast_to(qn[:, None, :], (BLKH, K, 1)).reshape(M, 1)
        xe = jnp.broadcast_to(xq_ref[...][:, None, :], (BLKH, K, 128)).reshape(M, 128)

        d = (xe - xg_ref[...]).astype(jnp.bfloat16)
        s1 = jnp.maximum(
            jnp.dot(d, dw1_ref[...], preferred_element_type=jnp.float32)
            + db1_ref[...], 0.0).astype(jnp.bfloat16)
        pos = (jnp.dot(s1, dw2_ref[...], preferred_element_type=jnp.float32)
               + db2_ref[...])

        num = jnp.sum(qe * kf32, axis=1, keepdims=True)
        kn = jnp.maximum(jnp.sqrt(jnp.sum(kf32 * kf32, axis=1, keepdims=True)), 1e-8)
        sim = num / (qne * kn)

        qmk = (qe - kf32).astype(jnp.bfloat16)
        rel = (sim * sw0_ref[...]
               + jnp.dot(qmk, swT_ref[...], preferred_element_type=jnp.float32)
               + sb_ref[...])
        h = (rel + pos).astype(jnp.bfloat16)
        a1 = jnp.maximum(
            jnp.dot(h, gw1_ref[...], preferred_element_type=jnp.float32)
            + gb1_ref[...], 0.0).astype(jnp.bfloat16)
        logits = (jnp.dot(a1, gw2_ref[...], preferred_element_type=jnp.float32)
                  + gb2_ref[...])
        l3 = (logits * inv_scale).reshape(BLKH, K, DM)
        m = jnp.max(l3, axis=1, keepdims=True)
        e = jnp.exp(l3 - m)
        s = jnp.sum(e, axis=1, keepdims=True)
        attn3 = e / s
        attn_ref[...] = attn3.reshape(M, DM)

        ve = (vf32 + pos).reshape(BLKH, K, DM)
        res = jnp.sum(attn3 * ve, axis=1)
        res_ref[...] = (jnp.dot(res, fc2_ref[...], precision=_HI)
                        + fb_ref[...] + pre_ref[...])

    grid = (NR // BLKH,)
    wspec = lambda shape: pl.BlockSpec(shape, lambda i: tuple(0 for _ in shape))
    return pl.pallas_call(
        body,
        grid=grid,
        in_specs=[
            pl.BlockSpec((M, DM), lambda i: (i, 0)),
            pl.BlockSpec((M, 128), lambda i: (i, 0)),
            pl.BlockSpec((BLKH, DM), lambda i: (i, 0)),
            pl.BlockSpec((BLKH, 128), lambda i: (i, 0)),
            pl.BlockSpec((BLKH, DP), lambda i: (i, 0)),
            wspec((128, DM)), wspec((1, DM)), wspec((DM, DM)), wspec((1, DM)),
            wspec((1, DM)), wspec((DM, DM)), wspec((1, DM)),
            wspec((DM, DM)), wspec((1, DM)), wspec((DM, DM)), wspec((1, DM)),
            wspec((DM, DP)), wspec((1, DP)),
        ],
        out_specs=[
            pl.BlockSpec((M, DM), lambda i: (i, 0)),
            pl.BlockSpec((BLKH, DP), lambda i: (i, 0)),
        ],
        out_shape=[
            jax.ShapeDtypeStruct((NR * K, DM), jnp.float32),
            jax.ShapeDtypeStruct((NR, DP), jnp.float32),
        ],
    )(kvg, xg, q, xq8, pre,
      dw1T, db1, dw2T, db2, sw0, swT, sim_b,
      gw1T, gb1, gw2T, gb2, fc2T, fc2_b)


def kernel(xyz, features, fc1_w, fc1_b, fc2_w, fc2_b, delta_w1, delta_b1,
           delta_w2, delta_b2, gamma_w1, gamma_b1, gamma_w2, gamma_b2,
           wq, wk, wv, sim_w, sim_b):
    feats = features.reshape(BN, DP)
    q, kvtab = _preproc(feats, fc1_w.T, fc1_b[None, :],
                        wq.T, wk.T, wv.T)

    xyz_flat = xyz.reshape(BN, 3)
    xyz8 = jnp.pad(xyz_flat, ((0, 0), (0, 5)))
    xyz_all_t = jnp.pad(xyz, ((0, 0), (0, 0), (0, 5))).transpose(0, 2, 1)
    xg_tab = jnp.pad(xyz_flat, ((0, 0), (0, 125)))

    dw1T = jnp.pad(delta_w1.T, ((0, 125), (0, 0))).astype(jnp.bfloat16)
    weights = (
        dw1T, delta_b1[None, :],
        delta_w2.T.astype(jnp.bfloat16), delta_b2[None, :],
        sim_w[:, 0][None, :], sim_w[:, 1:].T.astype(jnp.bfloat16),
        sim_b[None, :],
        gamma_w1.T.astype(jnp.bfloat16), gamma_b1[None, :],
        gamma_w2.T.astype(jnp.bfloat16), gamma_b2[None, :],
        fc2_w.T, fc2_b[None, :])

    attn_parts, res_parts = [], []
    for b in range(B):
        rows = slice(b * N, (b + 1) * N)
        idx_b = _knn(xyz8[rows], xyz_all_t[b:b + 1], b * N)
        kvg, xg = _gather_sc(kvtab, xg_tab, idx_b.reshape(N * K))
        attn_b, res_b = _heavy(kvg, xg, q[rows], xg_tab[rows], feats[rows],
                               *weights)
        attn_parts.append(attn_b.reshape(1, N, K, DM))
        res_parts.append(res_b.reshape(1, N, DP))

    res = jnp.concatenate(res_parts, axis=0)
    attn = jnp.concatenate(attn_parts, axis=0)
    return (res, attn)


# speedup vs baseline: 11.4570x; 1.1714x over previous
import functools
import math

import jax
import jax.numpy as jnp
from jax.experimental import pallas as pl
from jax.experimental.pallas import tpu as pltpu
from jax.experimental.pallas import tpu_sc as plsc

B, N = 2, 2048
DP, DM, K = 128, 512, 16
BN = B * N
NIDX = BN * K

BLKP = 512
BLKQ = 256
BLKH = 64
_HI = jax.lax.Precision.HIGHEST


def _preproc(features, fc1_wT, fc1_b, wqT, wkT, wvT):
    def body(f_ref, w1_ref, b1_ref, wq_ref, wk_ref, wv_ref,
             q_ref, kv_ref):
        x = jnp.dot(f_ref[...], w1_ref[...]) + b1_ref[...]
        q_ref[...] = jnp.dot(x, wq_ref[...])
        k = jnp.dot(x, wk_ref[...])
        v = jnp.dot(x, wv_ref[...])

        def _rne_hi(t):
            u = jax.lax.bitcast_convert_type(t, jnp.uint32)
            r = u + jnp.uint32(0x7FFF) + ((u >> 16) & jnp.uint32(1))
            return r & jnp.uint32(0xFFFF0000)

        packed = _rne_hi(k) | (_rne_hi(v) >> 16)
        kv_ref[...] = jax.lax.bitcast_convert_type(packed, jnp.int32)

    grid = (BN // BLKP,)
    return pl.pallas_call(
        body,
        grid=grid,
        in_specs=[
            pl.BlockSpec((BLKP, DP), lambda i: (i, 0)),
            pl.BlockSpec((DP, DM), lambda i: (0, 0)),
            pl.BlockSpec((1, DM), lambda i: (0, 0)),
            pl.BlockSpec((DM, DM), lambda i: (0, 0)),
            pl.BlockSpec((DM, DM), lambda i: (0, 0)),
            pl.BlockSpec((DM, DM), lambda i: (0, 0)),
        ],
        out_specs=[
            pl.BlockSpec((BLKP, DM), lambda i: (i, 0)),
            pl.BlockSpec((BLKP, DM), lambda i: (i, 0)),
        ],
        out_shape=[
            jax.ShapeDtypeStruct((BN, DM), jnp.float32),
            jax.ShapeDtypeStruct((BN, DM), jnp.int32),
        ],
    )(features, fc1_wT, fc1_b, wqT, wkT, wvT)


def _knn(xyz8, xyz_all_t, base_row):

    def body(xq_ref, xa_ref, idx_ref):
        xq = xq_ref[...]
        xa = xa_ref[0]
        qs2 = jnp.sum(xq * xq, axis=1, keepdims=True)
        ps2 = jnp.sum(xa * xa, axis=0, keepdims=True)
        qp = jnp.dot(xq, xa, precision=jax.lax.Precision.DEFAULT)
        dist = (qs2 + ps2) - 2.0 * qp
        iota = jax.lax.broadcasted_iota(jnp.int32, (BLKQ, N), 1)
        kiota = jax.lax.broadcasted_iota(jnp.int32, (BLKQ, K), 1)
        sel = jnp.zeros((BLKQ, K), jnp.int32)
        d = dist
        for j in range(K):
            mv = jnp.min(d, axis=1, keepdims=True)
            idx = jnp.min(jnp.where(d == mv, iota, N), axis=1, keepdims=True)
            sel = jnp.where(kiota == j, idx, sel)
            d = jnp.where(iota == idx, jnp.inf, d)
        idx_ref[...] = sel + base_row

    return pl.pallas_call(
        body,
        grid=(N // BLKQ,),
        in_specs=[
            pl.BlockSpec((BLKQ, 8), lambda g: (g, 0)),
            pl.BlockSpec((1, 8, N), lambda g: (0, 0, 0)),
        ],
        out_specs=pl.BlockSpec((BLKQ, K), lambda g: (g, 0)),
        out_shape=jax.ShapeDtypeStruct((N, K), jnp.int32),
    )(xyz8, xyz_all_t)


def _gather_sc(kv, xg, idx_flat):
    NI = idx_flat.shape[0]
    NC, NS = 2, 16
    NW = NC * NS
    BPW = NI // NW
    C = 128
    S = BPW // C
    mesh = plsc.VectorSubcoreMesh(core_axis_name="c", subcore_axis_name="s")

    @functools.partial(
        pl.kernel,
        out_type=(
            jax.ShapeDtypeStruct((NI, DM), jnp.int32),
            jax.ShapeDtypeStruct((NI, 128), jnp.float32),
        ),
        mesh=mesh,
        scratch_types=[
            pltpu.VMEM((C,), jnp.int32),
            pltpu.VMEM((C, DM), jnp.int32),
            pltpu.VMEM((C, 128), jnp.float32),
            pltpu.SemaphoreType.DMA,
            pltpu.SemaphoreType.DMA,
        ],
    )
    def kern(kv_hbm, xg_hbm, idx_hbm, kvo_hbm, xo_hbm,
             idxc, kvbuf, xbuf, sk, sx):
        wid = jax.lax.axis_index("s") * NC + jax.lax.axis_index("c")
        base = wid * BPW

        @pl.loop(0, S)
        def _(s):
            off = base + s * C
            pltpu.sync_copy(idx_hbm.at[pl.ds(off, C)], idxc)
            ck = pltpu.async_copy(kv_hbm.at[idxc], kvbuf, sk)
            cx = pltpu.async_copy(xg_hbm.at[idxc], xbuf, sx)
            ck.wait()
            cx.wait()
            pltpu.sync_copy(kvbuf, kvo_hbm.at[pl.ds(off, C)])
            pltpu.sync_copy(xbuf, xo_hbm.at[pl.ds(off, C)])

    return kern(kv, xg, idx_flat)


def _heavy(kvg, xg, q, xq8, pre,
           dw1T, db1, dw2T, db2, sw0, swT, sim_b,
           gw1T, gb1, gw2T, gb2, fc2T, fc2_b, prev=None, row_offset=0):
    NR = q.shape[0]
    M = BLKH * K
    OFF = row_offset // BLKH
    inv_scale = 1.0 / math.sqrt(float(DM))

    def body(kv_ref, xg_ref, q_ref, xq_ref, pre_ref,
             dw1_ref, db1_ref, dw2_ref, db2_ref, sw0_ref, swT_ref, sb_ref,
             gw1_ref, gb1_ref, gw2_ref, gb2_ref, fc2_ref, fb_ref,
             *rest):
        attn_ref, res_ref = rest[-2], rest[-1]
        ku = jax.lax.bitcast_convert_type(kv_ref[...], jnp.uint32)
        kf32 = jax.lax.bitcast_convert_type(ku & jnp.uint32(0xFFFF0000),
                                            jnp.float32)
        vf32 = jax.lax.bitcast_convert_type(ku << 16, jnp.float32)
        q = q_ref[...]
        qn = jnp.maximum(jnp.sqrt(jnp.sum(q * q, axis=1, keepdims=True)), 1e-8)
        qe = jnp.broadcast_to(q[:, None, :], (BLKH, K, DM)).reshape(M, DM)
        qne = jnp.broadcast_to(qn[:, None, :], (BLKH, K, 1)).reshape(M, 1)
        xe = jnp.broadcast_to(xq_ref[...][:, None, :], (BLKH, K, 128)).reshape(M, 128)

        d = (xe - xg_ref[...]).astype(jnp.bfloat16)
        s1 = jnp.maximum(
            jnp.dot(d, dw1_ref[...], preferred_element_type=jnp.float32)
            + db1_ref[...], 0.0).astype(jnp.bfloat16)
        pos = (jnp.dot(s1, dw2_ref[...], preferred_element_type=jnp.float32)
               + db2_ref[...])

        num = jnp.sum(qe * kf32, axis=1, keepdims=True)
        kn = jnp.maximum(jnp.sqrt(jnp.sum(kf32 * kf32, axis=1, keepdims=True)), 1e-8)
        sim = num / (qne * kn)

        qmk = (qe - kf32).astype(jnp.bfloat16)
        rel = (sim * sw0_ref[...]
               + jnp.dot(qmk, swT_ref[...], preferred_element_type=jnp.float32)
               + sb_ref[...])
        h = (rel + pos).astype(jnp.bfloat16)
        a1 = jnp.maximum(
            jnp.dot(h, gw1_ref[...], preferred_element_type=jnp.float32)
            + gb1_ref[...], 0.0).astype(jnp.bfloat16)
        logits = (jnp.dot(a1, gw2_ref[...], preferred_element_type=jnp.float32)
                  + gb2_ref[...])
        l3 = (logits * inv_scale).reshape(BLKH, K, DM)
        m = jnp.max(l3, axis=1, keepdims=True)
        e = jnp.exp(l3 - m)
        s = jnp.sum(e, axis=1, keepdims=True)
        attn3 = e * (1.0 / s)
        attn_ref[...] = attn3.reshape(M, DM)

        ve = (vf32 + pos).reshape(BLKH, K, DM)
        res = jnp.sum(attn3 * ve, axis=1)
        res_ref[...] = (jnp.dot(res, fc2_ref[...], precision=_HI)
                        + fb_ref[...] + pre_ref[...])

    grid = (NR // BLKH,)
    wspec = lambda shape: pl.BlockSpec(shape, lambda i: tuple(0 for _ in shape))
    in_specs = [
        pl.BlockSpec((M, DM), lambda i: (i, 0)),
        pl.BlockSpec((M, 128), lambda i: (i, 0)),
        pl.BlockSpec((BLKH, DM), lambda i: (i, 0)),
        pl.BlockSpec((BLKH, 128), lambda i: (i, 0)),
        pl.BlockSpec((BLKH, DP), lambda i: (i, 0)),
        wspec((128, DM)), wspec((1, DM)), wspec((DM, DM)), wspec((1, DM)),
        wspec((1, DM)), wspec((DM, DM)), wspec((1, DM)),
        wspec((DM, DM)), wspec((1, DM)), wspec((DM, DM)), wspec((1, DM)),
        wspec((DM, DP)), wspec((1, DP)),
    ]
    args = [kvg, xg, q, xq8, pre,
            dw1T, db1, dw2T, db2, sw0, swT, sim_b,
            gw1T, gb1, gw2T, gb2, fc2T, fc2_b]
    aliases = {}
    if prev is not None:
        anyspec = pl.BlockSpec(memory_space=pltpu.MemorySpace.HBM)
        in_specs += [anyspec, anyspec]
        args += [prev[0], prev[1]]
        aliases = {18: 0, 19: 1}
    return pl.pallas_call(
        body,
        grid=grid,
        in_specs=in_specs,
        out_specs=[
            pl.BlockSpec((M, DM), lambda i: (i + OFF, 0)),
            pl.BlockSpec((BLKH, DP), lambda i: (i + OFF, 0)),
        ],
        out_shape=[
            jax.ShapeDtypeStruct((NIDX, DM), jnp.float32),
            jax.ShapeDtypeStruct((BN, DP), jnp.float32),
        ],
        input_output_aliases=aliases,
    )(*args)


def kernel(xyz, features, fc1_w, fc1_b, fc2_w, fc2_b, delta_w1, delta_b1,
           delta_w2, delta_b2, gamma_w1, gamma_b1, gamma_w2, gamma_b2,
           wq, wk, wv, sim_w, sim_b):
    feats = features.reshape(BN, DP)
    q, kvtab = _preproc(feats, fc1_w.T, fc1_b[None, :],
                        wq.T, wk.T, wv.T)

    xyz_flat = xyz.reshape(BN, 3)
    xyz8 = jnp.pad(xyz_flat, ((0, 0), (0, 5)))
    xyz_all_t = jnp.pad(xyz, ((0, 0), (0, 0), (0, 5))).transpose(0, 2, 1)
    xg_tab = jnp.pad(xyz_flat, ((0, 0), (0, 125)))

    dw1T = jnp.pad(delta_w1.T, ((0, 125), (0, 0))).astype(jnp.bfloat16)
    weights = (
        dw1T, delta_b1[None, :],
        delta_w2.T.astype(jnp.bfloat16), delta_b2[None, :],
        sim_w[:, 0][None, :], sim_w[:, 1:].T.astype(jnp.bfloat16),
        sim_b[None, :],
        gamma_w1.T.astype(jnp.bfloat16), gamma_b1[None, :],
        gamma_w2.T.astype(jnp.bfloat16), gamma_b2[None, :],
        fc2_w.T, fc2_b[None, :])

    prev = None
    for b in range(B):
        rows = slice(b * N, (b + 1) * N)
        idx_b = _knn(xyz8[rows], xyz_all_t[b:b + 1], b * N)
        kvg, xg = _gather_sc(kvtab, xg_tab, idx_b.reshape(N * K))
        prev = _heavy(kvg, xg, q[rows], xg_tab[rows], feats[rows],
                      *weights, prev=prev, row_offset=b * N)

    attn_flat, res_flat = prev
    res = res_flat.reshape(B, N, DP)
    attn = attn_flat.reshape(B, N, K, DM)
    return (res, attn)


# speedup vs baseline: 12.1688x; 1.0621x over previous
import functools
import math

import jax
import jax.numpy as jnp
from jax.experimental import pallas as pl
from jax.experimental.pallas import tpu as pltpu
from jax.experimental.pallas import tpu_sc as plsc

B, N = 2, 2048
DP, DM, K = 128, 512, 16
BN = B * N
NIDX = BN * K

BLKP = 512
BLKQ = 256
BLKH = 128
_HI = jax.lax.Precision.HIGHEST


def _preproc(features, fc1_wT, fc1_b, wqT, wkT, wvT):
    def body(f_ref, w1_ref, b1_ref, wq_ref, wk_ref, wv_ref,
             q_ref, kv_ref, ikn_ref):
        x = jnp.dot(f_ref[...], w1_ref[...]) + b1_ref[...]
        q_ref[...] = jnp.dot(x, wq_ref[...])
        k = jnp.dot(x, wk_ref[...])
        v = jnp.dot(x, wv_ref[...])

        def _rne_hi(t):
            u = jax.lax.bitcast_convert_type(t, jnp.uint32)
            r = u + jnp.uint32(0x7FFF) + ((u >> 16) & jnp.uint32(1))
            return r & jnp.uint32(0xFFFF0000)

        packed = _rne_hi(k) | (_rne_hi(v) >> 16)
        kv_ref[...] = jax.lax.bitcast_convert_type(packed, jnp.int32)
        kn = jnp.maximum(jnp.sqrt(jnp.sum(k * k, axis=1, keepdims=True)), 1e-8)
        ikn_ref[...] = jnp.broadcast_to(1.0 / kn, (BLKP, 128))

    grid = (BN // BLKP,)
    return pl.pallas_call(
        body,
        grid=grid,
        in_specs=[
            pl.BlockSpec((BLKP, DP), lambda i: (i, 0)),
            pl.BlockSpec((DP, DM), lambda i: (0, 0)),
            pl.BlockSpec((1, DM), lambda i: (0, 0)),
            pl.BlockSpec((DM, DM), lambda i: (0, 0)),
            pl.BlockSpec((DM, DM), lambda i: (0, 0)),
            pl.BlockSpec((DM, DM), lambda i: (0, 0)),
        ],
        out_specs=[
            pl.BlockSpec((BLKP, DM), lambda i: (i, 0)),
            pl.BlockSpec((BLKP, DM), lambda i: (i, 0)),
            pl.BlockSpec((BLKP, 128), lambda i: (i, 0)),
        ],
        out_shape=[
            jax.ShapeDtypeStruct((BN, DM), jnp.float32),
            jax.ShapeDtypeStruct((BN, DM), jnp.int32),
            jax.ShapeDtypeStruct((BN, 128), jnp.float32),
        ],
    )(features, fc1_wT, fc1_b, wqT, wkT, wvT)


def _knn(xyz8, xyz_all_t, base_row):

    def body(xq_ref, xa_ref, idx_ref):
        xq = xq_ref[...]
        xa = xa_ref[0]
        qs2 = jnp.sum(xq * xq, axis=1, keepdims=True)
        ps2 = jnp.sum(xa * xa, axis=0, keepdims=True)
        qp = jnp.dot(xq, xa, precision=jax.lax.Precision.DEFAULT)
        dist = (qs2 + ps2) - 2.0 * qp
        iota = jax.lax.broadcasted_iota(jnp.int32, (BLKQ, N), 1)
        kiota = jax.lax.broadcasted_iota(jnp.int32, (BLKQ, K), 1)
        sel = jnp.zeros((BLKQ, K), jnp.int32)
        d = dist
        for j in range(K):
            mv = jnp.min(d, axis=1, keepdims=True)
            idx = jnp.min(jnp.where(d == mv, iota, N), axis=1, keepdims=True)
            sel = jnp.where(kiota == j, idx, sel)
            d = jnp.where(iota == idx, jnp.inf, d)
        idx_ref[...] = sel + base_row

    return pl.pallas_call(
        body,
        grid=(N // BLKQ,),
        in_specs=[
            pl.BlockSpec((BLKQ, 8), lambda g: (g, 0)),
            pl.BlockSpec((1, 8, N), lambda g: (0, 0, 0)),
        ],
        out_specs=pl.BlockSpec((BLKQ, K), lambda g: (g, 0)),
        out_shape=jax.ShapeDtypeStruct((N, K), jnp.int32),
    )(xyz8, xyz_all_t)


def _gather_sc(kv, xg, idx_flat):
    NI = idx_flat.shape[0]
    NC, NS = 2, 16
    NW = NC * NS
    BPW = NI // NW
    C = 128
    S = BPW // C
    mesh = plsc.VectorSubcoreMesh(core_axis_name="c", subcore_axis_name="s")

    @functools.partial(
        pl.kernel,
        out_type=(
            jax.ShapeDtypeStruct((NI, DM), jnp.int32),
            jax.ShapeDtypeStruct((NI, 128), jnp.float32),
        ),
        mesh=mesh,
        scratch_types=[
            pltpu.VMEM((C,), jnp.int32),
            pltpu.VMEM((C, DM), jnp.int32),
            pltpu.VMEM((C, 128), jnp.float32),
            pltpu.SemaphoreType.DMA,
            pltpu.SemaphoreType.DMA,
        ],
    )
    def kern(kv_hbm, xg_hbm, idx_hbm, kvo_hbm, xo_hbm,
             idxc, kvbuf, xbuf, sk, sx):
        wid = jax.lax.axis_index("s") * NC + jax.lax.axis_index("c")
        base = wid * BPW

        @pl.loop(0, S)
        def _(s):
            off = base + s * C
            pltpu.sync_copy(idx_hbm.at[pl.ds(off, C)], idxc)
            ck = pltpu.async_copy(kv_hbm.at[idxc], kvbuf, sk)
            cx = pltpu.async_copy(xg_hbm.at[idxc], xbuf, sx)
            ck.wait()
            cx.wait()
            pltpu.sync_copy(kvbuf, kvo_hbm.at[pl.ds(off, C)])
            pltpu.sync_copy(xbuf, xo_hbm.at[pl.ds(off, C)])

    return kern(kv, xg, idx_flat)


def _heavy(kvg, xg, q, xq8, pre,
           dw1T, db1, dw2T, db2, sw0, swT, sim_b,
           gw1T, gb1, gw2T, gb2, fc2T, fc2_b, prev=None, row_offset=0):
    NR = q.shape[0]
    M = BLKH * K
    OFF = row_offset // BLKH
    inv_scale = 1.0 / math.sqrt(float(DM))

    def body(kv_ref, xg_ref, q_ref, xq_ref, pre_ref,
             dw1_ref, db1_ref, dw2_ref, db2_ref, sw0_ref, swT_ref, sb_ref,
             gw1_ref, gb1_ref, gw2_ref, gb2_ref, fc2_ref, fb_ref,
             *rest):
        attn_ref, res_ref = rest[-2], rest[-1]
        ku = jax.lax.bitcast_convert_type(kv_ref[...], jnp.uint32)
        kf32 = jax.lax.bitcast_convert_type(ku & jnp.uint32(0xFFFF0000),
                                            jnp.float32)
        vf32 = jax.lax.bitcast_convert_type(ku << 16, jnp.float32)
        q = q_ref[...]
        qn = jnp.maximum(jnp.sqrt(jnp.sum(q * q, axis=1, keepdims=True)), 1e-8)
        qe = jnp.broadcast_to(q[:, None, :], (BLKH, K, DM)).reshape(M, DM)
        qne = jnp.broadcast_to(qn[:, None, :], (BLKH, K, 1)).reshape(M, 1)
        xe = jnp.broadcast_to(xq_ref[...][:, None, :], (BLKH, K, 128)).reshape(M, 128)

        d = (xe - xg_ref[...]).astype(jnp.bfloat16)
        s1 = jnp.maximum(
            jnp.dot(d, dw1_ref[...], preferred_element_type=jnp.float32)
            + db1_ref[...], 0.0).astype(jnp.bfloat16)
        pos = (jnp.dot(s1, dw2_ref[...], preferred_element_type=jnp.float32)
               + db2_ref[...])

        num = jnp.sum(qe * kf32, axis=1, keepdims=True)
        ikn = xg_ref[...][:, 3:4]
        sim = num * ((1.0 / qne) * ikn)

        qmk = (qe - kf32).astype(jnp.bfloat16)
        rel = (sim * sw0_ref[...]
               + jnp.dot(qmk, swT_ref[...], preferred_element_type=jnp.float32)
               + sb_ref[...])
        h = (rel + pos).astype(jnp.bfloat16)
        a1 = jnp.maximum(
            jnp.dot(h, gw1_ref[...], preferred_element_type=jnp.float32)
            + gb1_ref[...], 0.0).astype(jnp.bfloat16)
        logits = (jnp.dot(a1, gw2_ref[...], preferred_element_type=jnp.float32)
                  + gb2_ref[...])
        l3 = (logits * inv_scale).reshape(BLKH, K, DM)
        e = jnp.exp(l3)
        s = jnp.sum(e, axis=1, keepdims=True)
        attn3 = e * (1.0 / s)
        attn_ref[...] = attn3.reshape(M, DM)

        ve = (vf32 + pos).reshape(BLKH, K, DM)
        res = jnp.sum(attn3 * ve, axis=1)
        res_ref[...] = (jnp.dot(res, fc2_ref[...], precision=_HI)
                        + fb_ref[...] + pre_ref[...])

    grid = (NR // BLKH,)
    wspec = lambda shape: pl.BlockSpec(shape, lambda i: tuple(0 for _ in shape))
    in_specs = [
        pl.BlockSpec((M, DM), lambda i: (i, 0)),
        pl.BlockSpec((M, 128), lambda i: (i, 0)),
        pl.BlockSpec((BLKH, DM), lambda i: (i, 0)),
        pl.BlockSpec((BLKH, 128), lambda i: (i, 0)),
        pl.BlockSpec((BLKH, DP), lambda i: (i, 0)),
        wspec((128, DM)), wspec((1, DM)), wspec((DM, DM)), wspec((1, DM)),
        wspec((1, DM)), wspec((DM, DM)), wspec((1, DM)),
        wspec((DM, DM)), wspec((1, DM)), wspec((DM, DM)), wspec((1, DM)),
        wspec((DM, DP)), wspec((1, DP)),
    ]
    args = [kvg, xg, q, xq8, pre,
            dw1T, db1, dw2T, db2, sw0, swT, sim_b,
            gw1T, gb1, gw2T, gb2, fc2T, fc2_b]
    aliases = {}
    if prev is not None:
        anyspec = pl.BlockSpec(memory_space=pltpu.MemorySpace.HBM)
        in_specs += [anyspec, anyspec]
        args += [prev[0], prev[1]]
        aliases = {18: 0, 19: 1}
    return pl.pallas_call(
        body,
        grid=grid,
        in_specs=in_specs,
        out_specs=[
            pl.BlockSpec((M, DM), lambda i: (i + OFF, 0)),
            pl.BlockSpec((BLKH, DP), lambda i: (i + OFF, 0)),
        ],
        out_shape=[
            jax.ShapeDtypeStruct((NIDX, DM), jnp.float32),
            jax.ShapeDtypeStruct((BN, DP), jnp.float32),
        ],
        input_output_aliases=aliases,
    )(*args)


def kernel(xyz, features, fc1_w, fc1_b, fc2_w, fc2_b, delta_w1, delta_b1,
           delta_w2, delta_b2, gamma_w1, gamma_b1, gamma_w2, gamma_b2,
           wq, wk, wv, sim_w, sim_b):
    feats = features.reshape(BN, DP)
    q, kvtab, ikn_tab = _preproc(feats, fc1_w.T, fc1_b[None, :],
                                 wq.T, wk.T, wv.T)

    xyz_flat = xyz.reshape(BN, 3)
    xyz8 = jnp.pad(xyz_flat, ((0, 0), (0, 5)))
    xyz_all_t = jnp.pad(xyz, ((0, 0), (0, 0), (0, 5))).transpose(0, 2, 1)
    xg_tab = jnp.concatenate(
        [xyz_flat, ikn_tab[:, :1], jnp.zeros((BN, 124), jnp.float32)], axis=1)

    dw1T = jnp.pad(delta_w1.T, ((0, 125), (0, 0))).astype(jnp.bfloat16)
    weights = (
        dw1T, delta_b1[None, :],
        delta_w2.T.astype(jnp.bfloat16), delta_b2[None, :],
        sim_w[:, 0][None, :], sim_w[:, 1:].T.astype(jnp.bfloat16),
        sim_b[None, :],
        gamma_w1.T.astype(jnp.bfloat16), gamma_b1[None, :],
        gamma_w2.T.astype(jnp.bfloat16), gamma_b2[None, :],
        fc2_w.T, fc2_b[None, :])

    prev = None
    for b in range(B):
        rows = slice(b * N, (b + 1) * N)
        idx_b = _knn(xyz8[rows], xyz_all_t[b:b + 1], b * N)
        kvg, xg = _gather_sc(kvtab, xg_tab, idx_b.reshape(N * K))
        prev = _heavy(kvg, xg, q[rows], xg_tab[rows], feats[rows],
                      *weights, prev=prev, row_offset=b * N)

    attn_flat, res_flat = prev
    res = res_flat.reshape(B, N, DP)
    attn = attn_flat.reshape(B, N, K, DM)
    return (res, attn)


# speedup vs baseline: 13.0626x; 1.0735x over previous
import functools
import math

import jax
import jax.numpy as jnp
from jax.experimental import pallas as pl
from jax.experimental.pallas import tpu as pltpu
from jax.experimental.pallas import tpu_sc as plsc

B, N = 2, 2048
DP, DM, K = 128, 512, 16
BN = B * N
NIDX = BN * K

BLKP = 512
BLKQ = 256
BLKH = 128
_HI = jax.lax.Precision.HIGHEST


def _preproc(features, xyzp, fc1_wT, fc1_b, wqT, wkT, wvT):
    def body(f_ref, xp_ref, w1_ref, b1_ref, wq_ref, wk_ref, wv_ref,
             q_ref, kv_ref, xt_ref):
        x = jnp.dot(f_ref[...], w1_ref[...]) + b1_ref[...]
        q_ref[...] = jnp.dot(x, wq_ref[...])
        k = jnp.dot(x, wk_ref[...])
        v = jnp.dot(x, wv_ref[...])

        def _rne_hi(t):
            u = jax.lax.bitcast_convert_type(t, jnp.uint32)
            r = u + jnp.uint32(0x7FFF) + ((u >> 16) & jnp.uint32(1))
            return r & jnp.uint32(0xFFFF0000)

        packed = _rne_hi(k) | (_rne_hi(v) >> 16)
        kv_ref[...] = jax.lax.bitcast_convert_type(packed, jnp.int32)
        kn = jnp.maximum(jnp.sqrt(jnp.sum(k * k, axis=1, keepdims=True)), 1e-8)
        lane = jax.lax.broadcasted_iota(jnp.int32, (BLKP, 128), 1)
        xt_ref[...] = jnp.where(lane == 3,
                                jnp.broadcast_to(1.0 / kn, (BLKP, 128)),
                                xp_ref[...])

    grid = (BN // BLKP,)
    return pl.pallas_call(
        body,
        grid=grid,
        in_specs=[
            pl.BlockSpec((BLKP, DP), lambda i: (i, 0)),
            pl.BlockSpec((BLKP, 128), lambda i: (i, 0)),
            pl.BlockSpec((DP, DM), lambda i: (0, 0)),
            pl.BlockSpec((1, DM), lambda i: (0, 0)),
            pl.BlockSpec((DM, DM), lambda i: (0, 0)),
            pl.BlockSpec((DM, DM), lambda i: (0, 0)),
            pl.BlockSpec((DM, DM), lambda i: (0, 0)),
        ],
        out_specs=[
            pl.BlockSpec((BLKP, DM), lambda i: (i, 0)),
            pl.BlockSpec((BLKP, DM), lambda i: (i, 0)),
            pl.BlockSpec((BLKP, 128), lambda i: (i, 0)),
        ],
        out_shape=[
            jax.ShapeDtypeStruct((BN, DM), jnp.float32),
            jax.ShapeDtypeStruct((BN, DM), jnp.int32),
            jax.ShapeDtypeStruct((BN, 128), jnp.float32),
        ],
    )(features, xyzp, fc1_wT, fc1_b, wqT, wkT, wvT)


def _knn(xyz8, xyz_all_t, base_row):

    def body(xq_ref, xa_ref, idx_ref):
        xq = xq_ref[...]
        xa = xa_ref[0]
        qs2 = jnp.sum(xq * xq, axis=1, keepdims=True)
        ps2 = jnp.sum(xa * xa, axis=0, keepdims=True)
        qp = jnp.dot(xq, xa, precision=jax.lax.Precision.DEFAULT)
        dist = (qs2 + ps2) - 2.0 * qp
        iota_f = jax.lax.broadcasted_iota(
            jnp.int32, (BLKQ, N), 1).astype(jnp.float32)
        kiota = jax.lax.broadcasted_iota(jnp.int32, (BLKQ, K), 1)
        sel = jnp.zeros((BLKQ, K), jnp.float32)
        d = dist
        for j in range(K):
            mv = jnp.min(d, axis=1, keepdims=True)
            c = jnp.where(d == mv, iota_f, 4096.0)
            idxf = jnp.min(c, axis=1, keepdims=True)
            sel = jnp.where(kiota == j, idxf, sel)
            d = jnp.where(c == idxf, jnp.inf, d)
        idx_ref[...] = sel.astype(jnp.int32) + base_row

    return pl.pallas_call(
        body,
        grid=(N // BLKQ,),
        in_specs=[
            pl.BlockSpec((BLKQ, 8), lambda g: (g, 0)),
            pl.BlockSpec((1, 8, N), lambda g: (0, 0, 0)),
        ],
        out_specs=pl.BlockSpec((BLKQ, K), lambda g: (g, 0)),
        out_shape=jax.ShapeDtypeStruct((N, K), jnp.int32),
    )(xyz8, xyz_all_t)


def _gather_sc(kv, xg, idx_flat):
    NI = idx_flat.shape[0]
    NC, NS = 2, 16
    NW = NC * NS
    BPW = NI // NW
    C = 128
    S = BPW // C
    mesh = plsc.VectorSubcoreMesh(core_axis_name="c", subcore_axis_name="s")

    @functools.partial(
        pl.kernel,
        out_type=(
            jax.ShapeDtypeStruct((NI, DM), jnp.int32),
            jax.ShapeDtypeStruct((NI, 128), jnp.float32),
        ),
        mesh=mesh,
        scratch_types=[
            pltpu.VMEM((C,), jnp.int32),
            pltpu.VMEM((C, DM), jnp.int32),
            pltpu.VMEM((C, 128), jnp.float32),
            pltpu.SemaphoreType.DMA,
            pltpu.SemaphoreType.DMA,
        ],
    )
    def kern(kv_hbm, xg_hbm, idx_hbm, kvo_hbm, xo_hbm,
             idxc, kvbuf, xbuf, sk, sx):
        wid = jax.lax.axis_index("s") * NC + jax.lax.axis_index("c")
        base = wid * BPW

        @pl.loop(0, S)
        def _(s):
            off = base + s * C
            pltpu.sync_copy(idx_hbm.at[pl.ds(off, C)], idxc)
            ck = pltpu.async_copy(kv_hbm.at[idxc], kvbuf, sk)
            cx = pltpu.async_copy(xg_hbm.at[idxc], xbuf, sx)
            ck.wait()
            cx.wait()
            pltpu.sync_copy(kvbuf, kvo_hbm.at[pl.ds(off, C)])
            pltpu.sync_copy(xbuf, xo_hbm.at[pl.ds(off, C)])

    return kern(kv, xg, idx_flat)


def _heavy(kvg, xg, q, xq8, pre,
           dw1T, db1, dw2T, db2, sw0, swT, sim_b,
           gw1T, gb1, gw2T, gb2, fc2T, fc2_b, prev=None, row_offset=0):
    NR = q.shape[0]
    M = BLKH * K
    OFF = row_offset // BLKH
    inv_scale = 1.0 / math.sqrt(float(DM))

    def body(kv_ref, xg_ref, q_ref, xq_ref, pre_ref,
             dw1_ref, db1_ref, dw2_ref, db2_ref, sw0_ref, swT_ref, sb_ref,
             gw1_ref, gb1_ref, gw2_ref, gb2_ref, fc2_ref, fb_ref,
             *rest):
        attn_ref, res_ref = rest[-2], rest[-1]
        ku = jax.lax.bitcast_convert_type(kv_ref[...], jnp.uint32)
        kf32 = jax.lax.bitcast_convert_type(ku & jnp.uint32(0xFFFF0000),
                                            jnp.float32)
        vf32 = jax.lax.bitcast_convert_type(ku << 16, jnp.float32)
        q = q_ref[...]
        qn = jnp.maximum(jnp.sqrt(jnp.sum(q * q, axis=1, keepdims=True)), 1e-8)
        qe = jnp.broadcast_to(q[:, None, :], (BLKH, K, DM)).reshape(M, DM)
        qne = jnp.broadcast_to(qn[:, None, :], (BLKH, K, 1)).reshape(M, 1)
        xe = jnp.broadcast_to(xq_ref[...][:, None, :], (BLKH, K, 128)).reshape(M, 128)

        d = (xe - xg_ref[...]).astype(jnp.bfloat16)
        s1 = jnp.maximum(
            jnp.dot(d, dw1_ref[...], preferred_element_type=jnp.float32)
            + db1_ref[...], 0.0).astype(jnp.bfloat16)
        pos = (jnp.dot(s1, dw2_ref[...], preferred_element_type=jnp.float32)
               + db2_ref[...])

        num = jnp.sum(qe * kf32, axis=1, keepdims=True)
        ikn = xg_ref[...][:, 3:4]
        sim = num * ((1.0 / qne) * ikn)

        qmk = (qe - kf32).astype(jnp.bfloat16)
        rel = (sim * sw0_ref[...]
               + jnp.dot(qmk, swT_ref[...], preferred_element_type=jnp.float32)
               + sb_ref[...])
        h = (rel + pos).astype(jnp.bfloat16)
        a1 = jnp.maximum(
            jnp.dot(h, gw1_ref[...], preferred_element_type=jnp.float32)
            + gb1_ref[...], 0.0).astype(jnp.bfloat16)
        logits = (jnp.dot(a1, gw2_ref[...], preferred_element_type=jnp.float32)
                  + gb2_ref[...])
        l3 = (logits * inv_scale).reshape(BLKH, K, DM)
        e = jnp.exp(l3)
        s = jnp.sum(e, axis=1, keepdims=True)
        attn3 = e * (1.0 / s)
        attn_ref[...] = attn3.reshape(M, DM)

        ve = (vf32 + pos).reshape(BLKH, K, DM)
        res = jnp.sum(attn3 * ve, axis=1)
        res_ref[...] = (jnp.dot(res, fc2_ref[...], precision=_HI)
                        + fb_ref[...] + pre_ref[...])

    grid = (NR // BLKH,)
    wspec = lambda shape: pl.BlockSpec(shape, lambda i: tuple(0 for _ in shape))
    in_specs = [
        pl.BlockSpec((M, DM), lambda i: (i, 0)),
        pl.BlockSpec((M, 128), lambda i: (i, 0)),
        pl.BlockSpec((BLKH, DM), lambda i: (i, 0)),
        pl.BlockSpec((BLKH, 128), lambda i: (i, 0)),
        pl.BlockSpec((BLKH, DP), lambda i: (i, 0)),
        wspec((128, DM)), wspec((1, DM)), wspec((DM, DM)), wspec((1, DM)),
        wspec((1, DM)), wspec((DM, DM)), wspec((1, DM)),
        wspec((DM, DM)), wspec((1, DM)), wspec((DM, DM)), wspec((1, DM)),
        wspec((DM, DP)), wspec((1, DP)),
    ]
    args = [kvg, xg, q, xq8, pre,
            dw1T, db1, dw2T, db2, sw0, swT, sim_b,
            gw1T, gb1, gw2T, gb2, fc2T, fc2_b]
    aliases = {}
    if prev is not None:
        anyspec = pl.BlockSpec(memory_space=pltpu.MemorySpace.HBM)
        in_specs += [anyspec, anyspec]
        args += [prev[0], prev[1]]
        aliases = {18: 0, 19: 1}
    return pl.pallas_call(
        body,
        grid=grid,
        in_specs=in_specs,
        out_specs=[
            pl.BlockSpec((M, DM), lambda i: (i + OFF, 0)),
            pl.BlockSpec((BLKH, DP), lambda i: (i + OFF, 0)),
        ],
        out_shape=[
            jax.ShapeDtypeStruct((NIDX, DM), jnp.float32),
            jax.ShapeDtypeStruct((BN, DP), jnp.float32),
        ],
        input_output_aliases=aliases,
    )(*args)


def kernel(xyz, features, fc1_w, fc1_b, fc2_w, fc2_b, delta_w1, delta_b1,
           delta_w2, delta_b2, gamma_w1, gamma_b1, gamma_w2, gamma_b2,
           wq, wk, wv, sim_w, sim_b):
    feats = features.reshape(BN, DP)
    xyz_flat = xyz.reshape(BN, 3)
    q, kvtab, xg_tab = _preproc(feats, jnp.pad(xyz_flat, ((0, 0), (0, 125))),
                                fc1_w.T, fc1_b[None, :], wq.T, wk.T, wv.T)

    xyz8 = jnp.pad(xyz_flat, ((0, 0), (0, 5)))
    xyz_all_t = jnp.pad(xyz, ((0, 0), (0, 0), (0, 5))).transpose(0, 2, 1)


    dw1T = jnp.pad(delta_w1.T, ((0, 125), (0, 0))).astype(jnp.bfloat16)
    weights = (
        dw1T, delta_b1[None, :],
        delta_w2.T.astype(jnp.bfloat16), delta_b2[None, :],
        sim_w[:, 0][None, :], sim_w[:, 1:].T.astype(jnp.bfloat16),
        sim_b[None, :],
        gamma_w1.T.astype(jnp.bfloat16), gamma_b1[None, :],
        gamma_w2.T.astype(jnp.bfloat16), gamma_b2[None, :],
        fc2_w.T, fc2_b[None, :])

    prev = None
    for b in range(B):
        rows = slice(b * N, (b + 1) * N)
        idx_b = _knn(xyz8[rows], xyz_all_t[b:b + 1], b * N)
        kvg, xg = _gather_sc(kvtab, xg_tab, idx_b.reshape(N * K))
        prev = _heavy(kvg, xg, q[rows], xg_tab[rows], feats[rows],
                      *weights, prev=prev, row_offset=b * N)

    attn_flat, res_flat = prev
    res = res_flat.reshape(B, N, DP)
    attn = attn_flat.reshape(B, N, K, DM)
    return (res, attn)
